# Initial kernel scaffold; baseline (speedup 1.0000x reference)
import functools

import jax
import jax.numpy as jnp
from jax import lax
from jax.experimental import pallas as pl
from jax.experimental.pallas import tpu as pltpu
from jax.experimental.pallas import tpu_sc as plsc

N_SEG = 10000
N_TOK = 320000
D = 128

NC = 2
NS = 16
NW = NC * NS

TOK_PER_TILE = N_TOK // NW
CHUNK = 80
N_CHUNK = TOK_PER_TILE // CHUNK
ROWS_PER_TILE = N_SEG // NS
ZROWS = 125


def _sc_scatter_kernel(u_hbm, v_hbm, b_hbm, p_hbm, acc, vbuf, idxbuf, zbuf):
    cid = lax.axis_index("c")
    sid = lax.axis_index("s")
    wid = cid * NS + sid

    @pl.when(cid == 0)
    def _():
        pltpu.sync_copy(u_hbm.at[pl.ds(sid * ROWS_PER_TILE, ROWS_PER_TILE)],
                        acc.at[pl.ds(sid * ROWS_PER_TILE, ROWS_PER_TILE)])

    @pl.when(cid != 0)
    def _():
        def zero_body(i, _):
            r = i // (D // 16)
            g = i % (D // 16)
            zbuf[r, pl.ds(g * 16, 16)] = jnp.zeros((16,), jnp.float32)
            return 0
        lax.fori_loop(0, ZROWS * (D // 16), zero_body, 0)
        for j in range(ROWS_PER_TILE // ZROWS):
            pltpu.sync_copy(zbuf,
                            acc.at[pl.ds(sid * ROWS_PER_TILE + j * ZROWS, ZROWS)])

    plsc.subcore_barrier()

    def chunk_body(c, _):
        base = wid * TOK_PER_TILE + c * CHUNK
        pltpu.sync_copy(b_hbm.at[pl.ds(base, CHUNK)], idxbuf)
        pltpu.sync_copy(v_hbm.at[pl.ds(base, CHUNK)], vbuf)
        pltpu.sync_copy(vbuf, acc.at[idxbuf], add=True)
        return 0
    lax.fori_loop(0, N_CHUNK, chunk_body, 0)

    plsc.subcore_barrier()

    pltpu.sync_copy(acc.at[pl.ds(sid * ROWS_PER_TILE, ROWS_PER_TILE)],
                    p_hbm.at[cid, pl.ds(sid * ROWS_PER_TILE, ROWS_PER_TILE)])


def _combine_body(p_ref, o_ref):
    o_ref[...] = p_ref[0] + p_ref[1]


def kernel(u, v, batch):
    batch = batch.astype(jnp.int32)

    scatter = pl.kernel(
        _sc_scatter_kernel,
        out_type=jax.ShapeDtypeStruct((NC, N_SEG, D), jnp.float32),
        mesh=plsc.VectorSubcoreMesh(core_axis_name="c", subcore_axis_name="s"),
        scratch_types=[
            pltpu.VMEM_SHARED((N_SEG, D), jnp.float32),
            pltpu.VMEM((CHUNK, D), jnp.float32),
            pltpu.VMEM((CHUNK,), jnp.int32),
            pltpu.VMEM((ZROWS, D), jnp.float32),
        ],
    )
    p = scatter(u, v, batch)

    BLK = 1000
    return pl.pallas_call(
        _combine_body,
        grid=(N_SEG // BLK,),
        in_specs=[pl.BlockSpec((NC, BLK, D), lambda i: (0, i, 0))],
        out_specs=pl.BlockSpec((BLK, D), lambda i: (i, 0)),
        out_shape=jax.ShapeDtypeStruct((N_SEG, D), jnp.float32),
    )(p)


# speedup vs baseline: 3.7258x; 3.7258x over previous
import functools

import jax
import jax.numpy as jnp
from jax import lax
from jax.experimental import pallas as pl
from jax.experimental.pallas import tpu as pltpu
from jax.experimental.pallas import tpu_sc as plsc

N_SEG = 10000
N_TOK = 320000
D = 128

NC = 2
NS = 16
NW = NC * NS

TOK_PER_TILE = N_TOK // NW
CHUNK = 80
N_CHUNK = TOK_PER_TILE // CHUNK
ROWS_PER_TILE = 624
TAIL_BASE = NS * ROWS_PER_TILE
TAIL_ROWS = N_SEG - TAIL_BASE
ZROWS = 104


def _sc_scatter_kernel(u_hbm, v_hbm, b_hbm, p_hbm, acc, vbuf, idxbuf, zbuf):
    cid = lax.axis_index("c")
    sid = lax.axis_index("s")
    wid = cid * NS + sid

    @pl.when(cid == 0)
    def _():
        pltpu.sync_copy(u_hbm.at[pl.ds(sid * ROWS_PER_TILE, ROWS_PER_TILE)],
                        acc.at[pl.ds(sid * ROWS_PER_TILE, ROWS_PER_TILE)])

        @pl.when(sid == 0)
        def _():
            pltpu.sync_copy(u_hbm.at[pl.ds(TAIL_BASE, TAIL_ROWS)],
                            acc.at[pl.ds(TAIL_BASE, TAIL_ROWS)])

    @pl.when(cid != 0)
    def _():
        def zero_body(i, _):
            r = i // (D // 16)
            g = i % (D // 16)
            zbuf[r, pl.ds(g * 16, 16)] = jnp.zeros((16,), jnp.float32)
            return 0
        lax.fori_loop(0, ZROWS * (D // 16), zero_body, 0)
        for j in range(ROWS_PER_TILE // ZROWS):
            pltpu.sync_copy(zbuf,
                            acc.at[pl.ds(sid * ROWS_PER_TILE + j * ZROWS, ZROWS)])

        @pl.when(sid == 0)
        def _():
            pltpu.sync_copy(zbuf.at[pl.ds(0, TAIL_ROWS)],
                            acc.at[pl.ds(TAIL_BASE, TAIL_ROWS)])

    plsc.subcore_barrier()

    def chunk_body(c, _):
        base = wid * TOK_PER_TILE + c * CHUNK
        pltpu.sync_copy(b_hbm.at[pl.ds(base, CHUNK)], idxbuf)
        pltpu.sync_copy(v_hbm.at[pl.ds(base, CHUNK)], vbuf)
        pltpu.sync_copy(vbuf, acc.at[idxbuf], add=True)
        return 0
    lax.fori_loop(0, N_CHUNK, chunk_body, 0)

    plsc.subcore_barrier()

    pltpu.sync_copy(acc.at[pl.ds(sid * ROWS_PER_TILE, ROWS_PER_TILE)],
                    p_hbm.at[cid, pl.ds(sid * ROWS_PER_TILE, ROWS_PER_TILE)])

    @pl.when(sid == 0)
    def _():
        pltpu.sync_copy(acc.at[pl.ds(TAIL_BASE, TAIL_ROWS)],
                        p_hbm.at[cid, pl.ds(TAIL_BASE, TAIL_ROWS)])


def _combine_body(p_ref, o_ref):
    o_ref[...] = p_ref[0] + p_ref[1]


def kernel(u, v, batch):
    batch = batch.astype(jnp.int32)

    scatter = pl.kernel(
        _sc_scatter_kernel,
        out_type=jax.ShapeDtypeStruct((NC, N_SEG, D), jnp.float32),
        mesh=plsc.VectorSubcoreMesh(core_axis_name="c", subcore_axis_name="s"),
        scratch_types=[
            pltpu.VMEM_SHARED((N_SEG, D), jnp.float32),
            pltpu.VMEM((CHUNK, D), jnp.float32),
            pltpu.VMEM((CHUNK,), jnp.int32),
            pltpu.VMEM((ZROWS, D), jnp.float32),
        ],
    )
    p = scatter(u, v, batch)

    BLK = 1000
    return pl.pallas_call(
        _combine_body,
        grid=(N_SEG // BLK,),
        in_specs=[pl.BlockSpec((NC, BLK, D), lambda i: (0, i, 0))],
        out_specs=pl.BlockSpec((BLK, D), lambda i: (i, 0)),
        out_shape=jax.ShapeDtypeStruct((N_SEG, D), jnp.float32),
    )(p)


# speedup vs baseline: 7.5474x; 2.0258x over previous
import jax
import jax.numpy as jnp
from jax import lax
from jax.experimental import pallas as pl
from jax.experimental.pallas import tpu as pltpu
from jax.experimental.pallas import tpu_sc as plsc

N_SEG = 10000
N_TOK = 320000
D = 128

NC = 2
NS = 16
NW = NC * NS

TOK_PER_TILE = N_TOK // NW
CHUNK = 80
N_CHUNK = TOK_PER_TILE // CHUNK
NBUF = 4
NGROUP = (N_CHUNK - 1) // NBUF
ROWS_PER_TILE = 624
TAIL_BASE = NS * ROWS_PER_TILE
TAIL_ROWS = N_SEG - TAIL_BASE


def _sc_scatter_kernel(u_hbm, v_hbm, b_hbm, p_hbm, acc,
                       v0, v1, v2, v3, i0, i1, i2, i3,
                       lsem0, lsem1, lsem2, lsem3,
                       ssem0, ssem1, ssem2, ssem3):
    vbufs = (v0, v1, v2, v3)
    ibufs = (i0, i1, i2, i3)
    lsems = (lsem0, lsem1, lsem2, lsem3)
    ssems = (ssem0, ssem1, ssem2, ssem3)

    cid = lax.axis_index("c")
    sid = lax.axis_index("s")
    wid = cid * NS + sid

    @pl.when(cid == 0)
    def _():
        pltpu.sync_copy(u_hbm.at[pl.ds(sid * ROWS_PER_TILE, ROWS_PER_TILE)],
                        acc.at[pl.ds(sid * ROWS_PER_TILE, ROWS_PER_TILE)])

        @pl.when(sid == 0)
        def _():
            pltpu.sync_copy(u_hbm.at[pl.ds(TAIL_BASE, TAIL_ROWS)],
                            acc.at[pl.ds(TAIL_BASE, TAIL_ROWS)])

    @pl.when(cid != 0)
    def _():
        def zero_body(i, _):
            r = i // (D // 16)
            g = i % (D // 16)
            v0[r, pl.ds(g * 16, 16)] = jnp.zeros((16,), jnp.float32)
            return 0
        lax.fori_loop(0, CHUNK * (D // 16), zero_body, 0)
        for j in range(ROWS_PER_TILE // CHUNK):
            pltpu.sync_copy(v0,
                            acc.at[pl.ds(sid * ROWS_PER_TILE + j * CHUNK, CHUNK)])
        rem = ROWS_PER_TILE - (ROWS_PER_TILE // CHUNK) * CHUNK
        pltpu.sync_copy(v0.at[pl.ds(0, rem)],
                        acc.at[pl.ds(sid * ROWS_PER_TILE + ROWS_PER_TILE - rem,
                                     rem)])

        @pl.when(sid == 0)
        def _():
            pltpu.sync_copy(v0.at[pl.ds(0, TAIL_ROWS)],
                            acc.at[pl.ds(TAIL_BASE, TAIL_ROWS)])

    plsc.subcore_barrier()

    tok0 = wid * TOK_PER_TILE

    def start_load(c, b):
        base = tok0 + c * CHUNK
        pltpu.async_copy(b_hbm.at[pl.ds(base, CHUNK)], ibufs[b], lsems[b])
        pltpu.async_copy(v_hbm.at[pl.ds(base, CHUNK)], vbufs[b], lsems[b])

    def wait_load(c, b):
        base = tok0 + c * CHUNK
        pltpu.make_async_copy(b_hbm.at[pl.ds(base, CHUNK)], ibufs[b],
                              lsems[b]).wait()
        pltpu.make_async_copy(v_hbm.at[pl.ds(base, CHUNK)], vbufs[b],
                              lsems[b]).wait()

    for b in range(NBUF):
        start_load(b, b)

    def group_body(g, _):
        for b in range(NBUF):
            c = g * NBUF + b
            wait_load(c, b)
            pltpu.async_copy(vbufs[b], acc.at[ibufs[b]], ssems[b], add=True)
        for b in range(NBUF):
            pltpu.make_async_copy(vbufs[b], acc.at[ibufs[b]], ssems[b]).wait()

            @pl.when(g + 1 < NGROUP)
            def _():
                start_load((g + 1) * NBUF + b, b)
        return 0
    lax.fori_loop(0, NGROUP, group_body, 0)

    last = NGROUP * NBUF
    base = tok0 + last * CHUNK
    pltpu.sync_copy(b_hbm.at[pl.ds(base, CHUNK)], i0)
    pltpu.sync_copy(v_hbm.at[pl.ds(base, CHUNK)], v0)
    pltpu.sync_copy(v0, acc.at[i0], add=True)

    plsc.subcore_barrier()

    pltpu.sync_copy(acc.at[pl.ds(sid * ROWS_PER_TILE, ROWS_PER_TILE)],
                    p_hbm.at[cid, pl.ds(sid * ROWS_PER_TILE, ROWS_PER_TILE)])

    @pl.when(sid == 0)
    def _():
        pltpu.sync_copy(acc.at[pl.ds(TAIL_BASE, TAIL_ROWS)],
                        p_hbm.at[cid, pl.ds(TAIL_BASE, TAIL_ROWS)])


def _combine_body(p_ref, o_ref):
    o_ref[...] = p_ref[0] + p_ref[1]


def kernel(u, v, batch):
    batch = batch.astype(jnp.int32)

    scatter = pl.kernel(
        _sc_scatter_kernel,
        out_type=jax.ShapeDtypeStruct((NC, N_SEG, D), jnp.float32),
        mesh=plsc.VectorSubcoreMesh(core_axis_name="c", subcore_axis_name="s"),
        scratch_types=(
            [pltpu.VMEM_SHARED((N_SEG, D), jnp.float32)]
            + [pltpu.VMEM((CHUNK, D), jnp.float32) for _ in range(NBUF)]
            + [pltpu.VMEM((CHUNK,), jnp.int32) for _ in range(NBUF)]
            + [pltpu.SemaphoreType.DMA for _ in range(2 * NBUF)]
        ),
    )
    p = scatter(u, v, batch)

    BLK = 1000
    return pl.pallas_call(
        _combine_body,
        grid=(N_SEG // BLK,),
        in_specs=[pl.BlockSpec((NC, BLK, D), lambda i: (0, i, 0))],
        out_specs=pl.BlockSpec((BLK, D), lambda i: (i, 0)),
        out_shape=jax.ShapeDtypeStruct((N_SEG, D), jnp.float32),
    )(p)
